# Initial kernel scaffold; baseline (speedup 1.0000x reference)
import jax
import jax.numpy as jnp
from jax.experimental import pallas as pl


def kernel(fp, x, edge_attr, edge_index, batch, fc1_W, fc1_b, bn_g, bn_b, lin0_W, lin0_b, conv_W, conv_b, conv_We, conv_We_b, a_src, a_dst, a_e, lstm_W_ih, lstm_W_hh, lstm_b_ih, lstm_b_hh, lin1_W, lin1_b, lin2_W, lin2_b):
    raise NotImplementedError("write your pallas kernel here")


# speedup vs baseline: 6.7532x; 6.7532x over previous
import functools

import jax
import jax.numpy as jnp
from jax import lax
from jax.experimental import pallas as pl
from jax.experimental.pallas import tpu as pltpu
from jax.experimental.pallas import tpu_sc as plsc

N = 10000
E = 320000
B = 128
DIM = 256
MOL_IN = 15
B_IN = 14
FP_DIM = 1024
FP_LIN = 64
OUT = 2

NW = 32
EPW = E // NW
CH = 80
NCH = EPW // CH
NP = 10240
RPT = NP // 16


_RB = 2000


def _k0a_body(x_ref, w0_ref, b0_ref, wc_ref, bc_ref, asrc_ref, adst_ref,
              h0_ref, h1_ref, hs_ref, hd_ref):
    x = x_ref[...]
    out0 = lax.dot_general(x, w0_ref[...], (((1,), (1,)), ((), ())),
                           preferred_element_type=jnp.float32)
    out0 = jnp.maximum(out0 + b0_ref[...][None, :], 0.0)
    h = lax.dot_general(out0, wc_ref[...], (((1,), (1,)), ((), ())),
                        preferred_element_type=jnp.float32)
    h = h + bc_ref[...][None, :]
    h0_ref[...] = h[:, :128]
    h1_ref[...] = h[:, 128:]
    hs_ref[...] = jnp.sum(h * asrc_ref[...][None, :], axis=1)[:, None]
    hd_ref[...] = jnp.sum(h * adst_ref[...][None, :], axis=1)[:, None]


def _k0a(x, lin0_W, lin0_b, conv_W, conv_b, a_src, a_dst):
    grid = (N // _RB,)
    return pl.pallas_call(
        _k0a_body,
        grid=grid,
        in_specs=[
            pl.BlockSpec((_RB, MOL_IN), lambda i: (i, 0)),
            pl.BlockSpec((DIM, MOL_IN), lambda i: (0, 0)),
            pl.BlockSpec((DIM,), lambda i: (0,)),
            pl.BlockSpec((DIM, DIM), lambda i: (0, 0)),
            pl.BlockSpec((DIM,), lambda i: (0,)),
            pl.BlockSpec((DIM,), lambda i: (0,)),
            pl.BlockSpec((DIM,), lambda i: (0,)),
        ],
        out_specs=[
            pl.BlockSpec((_RB, 128), lambda i: (i, 0)),
            pl.BlockSpec((_RB, 128), lambda i: (i, 0)),
            pl.BlockSpec((_RB, 1), lambda i: (i, 0)),
            pl.BlockSpec((_RB, 1), lambda i: (i, 0)),
        ],
        out_shape=[
            jax.ShapeDtypeStruct((N, 128), jnp.float32),
            jax.ShapeDtypeStruct((N, 128), jnp.float32),
            jax.ShapeDtypeStruct((N, 1), jnp.float32),
            jax.ShapeDtypeStruct((N, 1), jnp.float32),
        ],
    )(x, lin0_W, lin0_b, conv_W, conv_b, a_src, a_dst)


_EB = 16000


def _k0b_body(eattr_ref, we_ref, ae_ref, web_ref, ea_ref):
    ae = ae_ref[...]
    v = jnp.sum(we_ref[...] * ae[:, None], axis=0)
    c = jnp.sum(ae * web_ref[...])
    ea_ref[...] = (jnp.sum(eattr_ref[...] * v[None, :], axis=1) + c)[:, None]


def _k0b(edge_attr, conv_We, a_e, conv_We_b):
    grid = (E // _EB,)
    return pl.pallas_call(
        _k0b_body,
        grid=grid,
        in_specs=[
            pl.BlockSpec((_EB, B_IN), lambda i: (i, 0)),
            pl.BlockSpec((DIM, B_IN), lambda i: (0, 0)),
            pl.BlockSpec((DIM,), lambda i: (0,)),
            pl.BlockSpec((DIM,), lambda i: (0,)),
        ],
        out_specs=pl.BlockSpec((_EB, 1), lambda i: (i, 0)),
        out_shape=jax.ShapeDtypeStruct((E, 1), jnp.float32),
    )(edge_attr, conv_We, a_e, conv_We_b)


def _k1_body(src_hbm, dst_hbm, ea_hbm, hs_hbm, hd_hbm,
             expv_hbm, denp_hbm,
             src_v, dst_v, ea_v, hs_v, hd_v, ex_v, den_v):
    cid = lax.axis_index("c")
    sid = lax.axis_index("s")
    wid = sid * 2 + cid
    base = wid * EPW
    pltpu.sync_copy(src_hbm.at[pl.ds(base, EPW)], src_v)
    pltpu.sync_copy(dst_hbm.at[pl.ds(base, EPW)], dst_v)
    pltpu.sync_copy(ea_hbm.at[pl.ds(base, EPW)], ea_v)
    pltpu.sync_copy(hs_hbm, hs_v)
    pltpu.sync_copy(hd_hbm, hd_v)

    def zero_body(i, c):
        den_v[pl.ds(i * 16, 16)] = jnp.zeros((16,), jnp.float32)
        return c

    lax.fori_loop(0, N // 16, zero_body, 0)

    def body(e, c):
        off = e * 16
        s16 = src_v[pl.ds(off, 16)]
        d16 = dst_v[pl.ds(off, 16)]
        a16 = ea_v[pl.ds(off, 16)]
        hsg = plsc.load_gather(hs_v, [s16])
        hdg = plsc.load_gather(hd_v, [d16])
        t = hsg + hdg + a16
        lg = jnp.maximum(t, t * 0.2)
        ex = jnp.exp(lg)
        ex_v[pl.ds(off, 16)] = ex
        plsc.addupdate_scatter(den_v, [d16], ex)
        return c

    lax.fori_loop(0, EPW // 16, body, 0)
    pltpu.sync_copy(ex_v, expv_hbm.at[pl.ds(base, EPW)])
    pltpu.sync_copy(den_v, denp_hbm.at[wid, 0])


def _k1(src, dst, ea, hs, hd):
    mesh = plsc.VectorSubcoreMesh(core_axis_name="c", subcore_axis_name="s")
    f = functools.partial(
        pl.kernel,
        mesh=mesh,
        compiler_params=pltpu.CompilerParams(needs_layout_passes=False),
        out_type=[
            jax.ShapeDtypeStruct((E,), jnp.float32),
            jax.ShapeDtypeStruct((NW, 1, N), jnp.float32),
        ],
        scratch_types=[
            pltpu.VMEM((EPW,), jnp.int32),
            pltpu.VMEM((EPW,), jnp.int32),
            pltpu.VMEM((EPW,), jnp.float32),
            pltpu.VMEM((N,), jnp.float32),
            pltpu.VMEM((N,), jnp.float32),
            pltpu.VMEM((EPW,), jnp.float32),
            pltpu.VMEM((N,), jnp.float32),
        ],
    )(_k1_body)
    return f(src, dst, ea, hs, hd)


def _k2_body(denp_ref, den_ref):
    den_ref[...] = jnp.sum(denp_ref[...], axis=0)


def _k2(denp):
    return pl.pallas_call(
        _k2_body,
        out_shape=jax.ShapeDtypeStruct((N,), jnp.float32),
    )(denp)


def _k2b_body(dst_hbm, ex_hbm, den_hbm, al_hbm, dst_v, ex_v, den_v, al_v):
    cid = lax.axis_index("c")
    sid = lax.axis_index("s")
    wid = sid * 2 + cid
    base = wid * EPW
    pltpu.sync_copy(dst_hbm.at[pl.ds(base, EPW)], dst_v)
    pltpu.sync_copy(ex_hbm.at[pl.ds(base, EPW)], ex_v)
    pltpu.sync_copy(den_hbm, den_v)

    def body(e, c):
        off = e * 16
        d16 = dst_v[pl.ds(off, 16)]
        deng = plsc.load_gather(den_v, [d16])
        al_v[pl.ds(off, 16)] = ex_v[pl.ds(off, 16)] / (deng + 1e-16)
        return c

    lax.fori_loop(0, EPW // 16, body, 0)
    pltpu.sync_copy(al_v, al_hbm.at[pl.ds(base, EPW)])


def _k2b(dst, expv, denom):
    mesh = plsc.VectorSubcoreMesh(core_axis_name="c", subcore_axis_name="s")
    f = functools.partial(
        pl.kernel,
        mesh=mesh,
        compiler_params=pltpu.CompilerParams(needs_layout_passes=False),
        out_type=jax.ShapeDtypeStruct((E,), jnp.float32),
        scratch_types=[
            pltpu.VMEM((EPW,), jnp.int32),
            pltpu.VMEM((EPW,), jnp.float32),
            pltpu.VMEM((N,), jnp.float32),
            pltpu.VMEM((EPW,), jnp.float32),
        ],
    )(_k2b_body)
    return f(dst, expv, denom)


def _k3_body(rec_hbm, h0_hbm, h1_hbm, zrows_hbm,
             aggp_hbm,
             rec_v, hbuf, sem, agg_s):
    cid = lax.axis_index("c")
    sid = lax.axis_index("s")
    wid = sid * 2 + cid
    tid = sid

    for half in range(2):
        h_hbm = h0_hbm if half == 0 else h1_hbm
        pltpu.sync_copy(zrows_hbm, agg_s.at[pl.ds(tid * RPT, RPT)])
        plsc.subcore_barrier()

        def chunk_body(j, c):
            pltpu.sync_copy(rec_hbm.at[wid, j], rec_v)
            cp = pltpu.async_copy(h_hbm.at[rec_v.at[0]], hbuf, sem)
            cp.wait()

            def grp_body(g, cc):
                av = plsc.bitcast(rec_v[2, pl.ds(g * 16, 16)], jnp.float32)
                for ri in range(16):
                    a_s = av[ri]
                    r = g * 16 + ri
                    for q in range(8):
                        off = q * 16
                        hbuf[r, pl.ds(off, 16)] = (
                            hbuf[r, pl.ds(off, 16)] * a_s)
                return cc

            lax.fori_loop(0, CH // 16, grp_body, 0)
            pltpu.sync_copy(hbuf, agg_s.at[rec_v.at[1]], add=True)
            return c

        lax.fori_loop(0, NCH, chunk_body, 0)
        plsc.subcore_barrier()
        pltpu.sync_copy(agg_s.at[pl.ds(tid * RPT, RPT)],
                        aggp_hbm.at[cid, half, pl.ds(tid * RPT, RPT)])
        plsc.subcore_barrier()


def _k3(rec, h0, h1, zrows):
    mesh = plsc.VectorSubcoreMesh(core_axis_name="c", subcore_axis_name="s")
    f = functools.partial(
        pl.kernel,
        mesh=mesh,
        compiler_params=pltpu.CompilerParams(needs_layout_passes=False),
        out_type=jax.ShapeDtypeStruct((2, 2, NP, 128), jnp.float32),
        scratch_types=[
            pltpu.VMEM((3, CH), jnp.int32),
            pltpu.VMEM((CH, 128), jnp.float32),
            pltpu.SemaphoreType.DMA,
            pltpu.VMEM_SHARED((NP, 128), jnp.float32),
        ],
    )(_k3_body)
    return f(rec, h0, h1, zrows)


def _k3c_body(rec_hbm, ea4_hbm, zrows_hbm, wsump_hbm,
              rec_v, e16, ebuf, ws_s):
    cid = lax.axis_index("c")
    sid = lax.axis_index("s")
    wid = sid * 2 + cid
    tid = sid

    def zb(i, c):
        for q in range(8):
            ebuf[i, pl.ds(q * 16, 16)] = jnp.zeros((16,), jnp.float32)
        return c

    lax.fori_loop(0, CH, zb, 0)
    pltpu.sync_copy(zrows_hbm, ws_s.at[pl.ds(tid * RPT, RPT)])
    plsc.subcore_barrier()

    def chunk_body(j, c):
        pltpu.sync_copy(rec_hbm.at[wid, j], rec_v)
        pltpu.sync_copy(ea4_hbm.at[wid, j], e16)

        def grp_body(g, cc):
            av = plsc.bitcast(rec_v[2, pl.ds(g * 16, 16)], jnp.float32)
            for ri in range(16):
                a_s = av[ri]
                r = g * 16 + ri
                ebuf[r, pl.ds(0, 16)] = e16[r, pl.ds(0, 16)] * a_s
            return cc

        lax.fori_loop(0, CH // 16, grp_body, 0)
        pltpu.sync_copy(ebuf, ws_s.at[rec_v.at[1]], add=True)
        return c

    lax.fori_loop(0, NCH, chunk_body, 0)
    plsc.subcore_barrier()
    pltpu.sync_copy(ws_s.at[pl.ds(tid * RPT, RPT)],
                    wsump_hbm.at[cid, pl.ds(tid * RPT, RPT)])


def _k3c(rec, ea4, zrows):
    mesh = plsc.VectorSubcoreMesh(core_axis_name="c", subcore_axis_name="s")
    f = functools.partial(
        pl.kernel,
        mesh=mesh,
        compiler_params=pltpu.CompilerParams(needs_layout_passes=False),
        out_type=jax.ShapeDtypeStruct((2, NP, 128), jnp.float32),
        scratch_types=[
            pltpu.VMEM((3, CH), jnp.int32),
            pltpu.VMEM((CH, 16), jnp.float32),
            pltpu.VMEM((CH, 128), jnp.float32),
            pltpu.VMEM_SHARED((NP, 128), jnp.float32),
        ],
    )(_k3c_body)
    return f(rec, ea4, zrows)


def _k4a_body(aggp_ref, wsump_ref, h0_ref, h1_ref, den_ref, we_ref, web_ref,
              out_ref):
    aggp = aggp_ref[...]
    agg = jnp.concatenate(
        [aggp[0, 0] + aggp[1, 0], aggp[0, 1] + aggp[1, 1]], axis=-1)
    wsum = (wsump_ref[...][0] + wsump_ref[...][1])[:, :B_IN]
    he = lax.dot_general(wsum, we_ref[...], (((1,), (1,)), ((), ())),
                         preferred_element_type=jnp.float32)
    den = den_ref[...][:, 0]
    asum = den / (den + 1e-16)
    h = jnp.concatenate([h0_ref[...], h1_ref[...]], axis=-1)
    out_ref[...] = jnp.maximum(
        agg + he + asum[:, None] * web_ref[...][None, :] + h, 0.0)


def _k4a(aggp, wsump, h0, h1, denom, conv_We, conv_We_b):
    grid = (N // _RB,)
    return pl.pallas_call(
        _k4a_body,
        grid=grid,
        in_specs=[
            pl.BlockSpec((2, 2, _RB, 128), lambda i: (0, 0, i, 0)),
            pl.BlockSpec((2, _RB, 128), lambda i: (0, i, 0)),
            pl.BlockSpec((_RB, 128), lambda i: (i, 0)),
            pl.BlockSpec((_RB, 128), lambda i: (i, 0)),
            pl.BlockSpec((_RB, 1), lambda i: (i, 0)),
            pl.BlockSpec((DIM, B_IN), lambda i: (0, 0)),
            pl.BlockSpec((DIM,), lambda i: (0,)),
        ],
        out_specs=pl.BlockSpec((_RB, DIM), lambda i: (i, 0)),
        out_shape=jax.ShapeDtypeStruct((N, DIM), jnp.float32),
    )(aggp, wsump, h0, h1, denom.reshape(N, 1), conv_We, conv_We_b)


def _k4b_body(out_ref, batch_ref, fp_ref, fc1W_ref, fc1b_ref, bng_ref,
              bnb_ref, wih_ref, whh_ref, bih_ref, bhh_ref, lin1W_ref,
              lin1b_ref, lin2W_ref, lin2b_ref, res_ref):
    outm = out_ref[...]
    bvec = batch_ref[...]
    gid = lax.broadcasted_iota(jnp.int32, (N, B), 1)
    mask = bvec[:, None] == gid

    q_star = jnp.zeros((B, 2 * DIM), jnp.float32)
    h_l = jnp.zeros((B, DIM), jnp.float32)
    c_l = jnp.zeros((B, DIM), jnp.float32)
    wih = wih_ref[...]
    whh = whh_ref[...]
    bih = bih_ref[...]
    bhh = bhh_ref[...]
    for _ in range(3):
        gates = (lax.dot_general(q_star, wih, (((1,), (1,)), ((), ())),
                                 preferred_element_type=jnp.float32)
                 + bih[None, :]
                 + lax.dot_general(h_l, whh, (((1,), (1,)), ((), ())),
                                   preferred_element_type=jnp.float32)
                 + bhh[None, :])
        i_g = gates[:, :DIM]
        f_g = gates[:, DIM:2 * DIM]
        g_g = gates[:, 2 * DIM:3 * DIM]
        o_g = gates[:, 3 * DIM:]
        c_l = jax.nn.sigmoid(f_g) * c_l + jax.nn.sigmoid(i_g) * jnp.tanh(g_g)
        h_l = jax.nn.sigmoid(o_g) * jnp.tanh(c_l)
        q = h_l
        m = lax.dot_general(outm, q, (((1,), (1,)), ((), ())),
                            preferred_element_type=jnp.float32)
        emax = jnp.max(jnp.where(mask, m, -1e30), axis=0)
        anum = jnp.where(mask, jnp.exp(m - emax[None, :]), 0.0)
        den = jnp.sum(anum, axis=0)
        amat = anum / (den + 1e-16)[None, :]
        r = lax.dot_general(amat, outm, (((0,), (0,)), ((), ())),
                            preferred_element_type=jnp.float32)
        q_star = jnp.concatenate([q, r], axis=-1)

    g_out = jnp.maximum(
        lax.dot_general(q_star, lin1W_ref[...], (((1,), (1,)), ((), ())),
                        preferred_element_type=jnp.float32)
        + lin1b_ref[...][None, :], 0.0)

    h_fp = lax.dot_general(fp_ref[...], fc1W_ref[...], (((1,), (1,)), ((), ())),
                           preferred_element_type=jnp.float32) + fc1b_ref[...][None, :]
    h_fp = h_fp / jnp.sqrt(1.0 + 1e-5) * bng_ref[...][None, :] + bnb_ref[...][None, :]
    out_fp = jnp.where(h_fp > 0, h_fp, jnp.exp(h_fp) - 1.0)

    cat = jnp.concatenate([g_out, out_fp], axis=-1)
    res_ref[...] = (lax.dot_general(cat, lin2W_ref[...], (((1,), (1,)), ((), ())),
                                    preferred_element_type=jnp.float32)
                    + lin2b_ref[...][None, :])


def _k4b(out, batch, fp, fc1_W, fc1_b, bn_g, bn_b, lstm_W_ih, lstm_W_hh,
         lstm_b_ih, lstm_b_hh, lin1_W, lin1_b, lin2_W, lin2_b):
    return pl.pallas_call(
        _k4b_body,
        out_shape=jax.ShapeDtypeStruct((B, OUT), jnp.float32),
    )(out, batch, fp, fc1_W, fc1_b, bn_g, bn_b, lstm_W_ih, lstm_W_hh,
      lstm_b_ih, lstm_b_hh, lin1_W, lin1_b, lin2_W, lin2_b)


def kernel(fp, x, edge_attr, edge_index, batch, fc1_W, fc1_b, bn_g, bn_b,
           lin0_W, lin0_b, conv_W, conv_b, conv_We, conv_We_b,
           a_src, a_dst, a_e, lstm_W_ih, lstm_W_hh, lstm_b_ih, lstm_b_hh,
           lin1_W, lin1_b, lin2_W, lin2_b):
    src = edge_index[0]
    dst = edge_index[1]

    h0, h1, hs2, hd2 = _k0a(x, lin0_W, lin0_b, conv_W, conv_b, a_src, a_dst)
    hs = hs2.reshape(N)
    hd = hd2.reshape(N)
    ea = _k0b(edge_attr, conv_We, a_e, conv_We_b).reshape(E)
    expv, denp = _k1(src, dst, ea, hs, hd)
    denom = _k2(denp.reshape(NW, N))
    alpha = _k2b(dst, expv, denom)

    rec = jnp.stack(
        [src.reshape(NW, NCH, CH),
         dst.reshape(NW, NCH, CH),
         lax.bitcast_convert_type(alpha, jnp.int32).reshape(NW, NCH, CH)],
        axis=2)
    ea4 = jnp.pad(edge_attr, ((0, 0), (0, 16 - B_IN))).reshape(NW, NCH, CH, 16)
    zrows = jnp.zeros((RPT, 128), jnp.float32)

    aggp = _k3(rec, h0, h1, zrows)
    wsump = _k3c(rec, ea4, zrows)
    out = _k4a(aggp, wsump, h0, h1, denom, conv_We, conv_We_b)
    return _k4b(out, batch, fp, fc1_W, fc1_b, bn_g, bn_b, lstm_W_ih,
                lstm_W_hh, lstm_b_ih, lstm_b_hh, lin1_W, lin1_b, lin2_W,
                lin2_b)


# speedup vs baseline: 7.8038x; 1.1556x over previous
import functools

import jax
import jax.numpy as jnp
from jax import lax
from jax.experimental import pallas as pl
from jax.experimental.pallas import tpu as pltpu
from jax.experimental.pallas import tpu_sc as plsc

N = 10000
E = 320000
B = 128
DIM = 256
MOL_IN = 15
B_IN = 14
FP_DIM = 1024
FP_LIN = 64
OUT = 2

NW = 32
EPW = E // NW
CH = 80
NCH = EPW // CH
NP = 10240
RPT = NP // 16


_RB = 2000


def _k0a_body(x_ref, w0_ref, b0_ref, wc_ref, bc_ref, asrc_ref, adst_ref,
              h0_ref, h1_ref, hs_ref, hd_ref):
    x = x_ref[...]
    out0 = lax.dot_general(x, w0_ref[...], (((1,), (1,)), ((), ())),
                           preferred_element_type=jnp.float32)
    out0 = jnp.maximum(out0 + b0_ref[...][None, :], 0.0)
    h = lax.dot_general(out0, wc_ref[...], (((1,), (1,)), ((), ())),
                        preferred_element_type=jnp.float32)
    h = h + bc_ref[...][None, :]
    h0_ref[...] = h[:, :128]
    h1_ref[...] = h[:, 128:]
    hs_ref[...] = jnp.sum(h * asrc_ref[...][None, :], axis=1)[:, None]
    hd_ref[...] = jnp.sum(h * adst_ref[...][None, :], axis=1)[:, None]


def _k0a(x, lin0_W, lin0_b, conv_W, conv_b, a_src, a_dst):
    grid = (N // _RB,)
    return pl.pallas_call(
        _k0a_body,
        grid=grid,
        in_specs=[
            pl.BlockSpec((_RB, MOL_IN), lambda i: (i, 0)),
            pl.BlockSpec((DIM, MOL_IN), lambda i: (0, 0)),
            pl.BlockSpec((DIM,), lambda i: (0,)),
            pl.BlockSpec((DIM, DIM), lambda i: (0, 0)),
            pl.BlockSpec((DIM,), lambda i: (0,)),
            pl.BlockSpec((DIM,), lambda i: (0,)),
            pl.BlockSpec((DIM,), lambda i: (0,)),
        ],
        out_specs=[
            pl.BlockSpec((_RB, 128), lambda i: (i, 0)),
            pl.BlockSpec((_RB, 128), lambda i: (i, 0)),
            pl.BlockSpec((_RB, 1), lambda i: (i, 0)),
            pl.BlockSpec((_RB, 1), lambda i: (i, 0)),
        ],
        out_shape=[
            jax.ShapeDtypeStruct((N, 128), jnp.float32),
            jax.ShapeDtypeStruct((N, 128), jnp.float32),
            jax.ShapeDtypeStruct((N, 1), jnp.float32),
            jax.ShapeDtypeStruct((N, 1), jnp.float32),
        ],
    )(x, lin0_W, lin0_b, conv_W, conv_b, a_src, a_dst)


_EB = 16000


def _k0b_body(eattr_ref, we_ref, ae_ref, web_ref, ea_ref):
    ae = ae_ref[...]
    v = jnp.sum(we_ref[...] * ae[:, None], axis=0)
    c = jnp.sum(ae * web_ref[...])
    ea_ref[...] = (jnp.sum(eattr_ref[...] * v[None, :], axis=1) + c)[:, None]


def _k0b(edge_attr, conv_We, a_e, conv_We_b):
    grid = (E // _EB,)
    return pl.pallas_call(
        _k0b_body,
        grid=grid,
        in_specs=[
            pl.BlockSpec((_EB, B_IN), lambda i: (i, 0)),
            pl.BlockSpec((DIM, B_IN), lambda i: (0, 0)),
            pl.BlockSpec((DIM,), lambda i: (0,)),
            pl.BlockSpec((DIM,), lambda i: (0,)),
        ],
        out_specs=pl.BlockSpec((_EB, 1), lambda i: (i, 0)),
        out_shape=jax.ShapeDtypeStruct((E, 1), jnp.float32),
    )(edge_attr, conv_We, a_e, conv_We_b)


def _k1_body(src_hbm, dst_hbm, ea_hbm, hs_hbm, hd_hbm,
             expv_hbm, denp_hbm,
             src_v, dst_v, ea_v, hs_v, hd_v, ex_v, den_v):
    cid = lax.axis_index("c")
    sid = lax.axis_index("s")
    wid = sid * 2 + cid
    base = wid * EPW
    pltpu.sync_copy(src_hbm.at[pl.ds(base, EPW)], src_v)
    pltpu.sync_copy(dst_hbm.at[pl.ds(base, EPW)], dst_v)
    pltpu.sync_copy(ea_hbm.at[pl.ds(base, EPW)], ea_v)
    pltpu.sync_copy(hs_hbm, hs_v)
    pltpu.sync_copy(hd_hbm, hd_v)

    def zero_body(i, c):
        den_v[pl.ds(i * 16, 16)] = jnp.zeros((16,), jnp.float32)
        return c

    lax.fori_loop(0, N // 16, zero_body, 0)

    def body(e, c):
        off = e * 16
        s16 = src_v[pl.ds(off, 16)]
        d16 = dst_v[pl.ds(off, 16)]
        a16 = ea_v[pl.ds(off, 16)]
        hsg = plsc.load_gather(hs_v, [s16])
        hdg = plsc.load_gather(hd_v, [d16])
        t = hsg + hdg + a16
        lg = jnp.maximum(t, t * 0.2)
        ex = jnp.exp(lg)
        ex_v[pl.ds(off, 16)] = ex
        plsc.addupdate_scatter(den_v, [d16], ex)
        return c

    lax.fori_loop(0, EPW // 16, body, 0)
    pltpu.sync_copy(ex_v, expv_hbm.at[pl.ds(base, EPW)])
    pltpu.sync_copy(den_v, denp_hbm.at[wid, 0])


def _k1(src, dst, ea, hs, hd):
    mesh = plsc.VectorSubcoreMesh(core_axis_name="c", subcore_axis_name="s")
    f = functools.partial(
        pl.kernel,
        mesh=mesh,
        compiler_params=pltpu.CompilerParams(needs_layout_passes=False),
        out_type=[
            jax.ShapeDtypeStruct((E,), jnp.float32),
            jax.ShapeDtypeStruct((NW, 1, N), jnp.float32),
        ],
        scratch_types=[
            pltpu.VMEM((EPW,), jnp.int32),
            pltpu.VMEM((EPW,), jnp.int32),
            pltpu.VMEM((EPW,), jnp.float32),
            pltpu.VMEM((N,), jnp.float32),
            pltpu.VMEM((N,), jnp.float32),
            pltpu.VMEM((EPW,), jnp.float32),
            pltpu.VMEM((N,), jnp.float32),
        ],
    )(_k1_body)
    return f(src, dst, ea, hs, hd)


def _k2_body(denp_ref, den_ref):
    den_ref[...] = jnp.sum(denp_ref[...], axis=0)


def _k2(denp):
    return pl.pallas_call(
        _k2_body,
        out_shape=jax.ShapeDtypeStruct((N,), jnp.float32),
    )(denp)


def _k2b_body(dst_hbm, ex_hbm, den_hbm, al_hbm, dst_v, ex_v, den_v, al_v):
    cid = lax.axis_index("c")
    sid = lax.axis_index("s")
    wid = sid * 2 + cid
    base = wid * EPW
    pltpu.sync_copy(dst_hbm.at[pl.ds(base, EPW)], dst_v)
    pltpu.sync_copy(ex_hbm.at[pl.ds(base, EPW)], ex_v)
    pltpu.sync_copy(den_hbm, den_v)

    def body(e, c):
        off = e * 16
        d16 = dst_v[pl.ds(off, 16)]
        deng = plsc.load_gather(den_v, [d16])
        al_v[pl.ds(off, 16)] = ex_v[pl.ds(off, 16)] / (deng + 1e-16)
        return c

    lax.fori_loop(0, EPW // 16, body, 0)
    pltpu.sync_copy(al_v, al_hbm.at[pl.ds(base, EPW)])


def _k2b(dst, expv, denom):
    mesh = plsc.VectorSubcoreMesh(core_axis_name="c", subcore_axis_name="s")
    f = functools.partial(
        pl.kernel,
        mesh=mesh,
        compiler_params=pltpu.CompilerParams(needs_layout_passes=False),
        out_type=jax.ShapeDtypeStruct((E,), jnp.float32),
        scratch_types=[
            pltpu.VMEM((EPW,), jnp.int32),
            pltpu.VMEM((EPW,), jnp.float32),
            pltpu.VMEM((N,), jnp.float32),
            pltpu.VMEM((EPW,), jnp.float32),
        ],
    )(_k2b_body)
    return f(dst, expv, denom)


def _scale_rows(hb, rec):
    def grp_body(g, cc):
        av = plsc.bitcast(rec[2, pl.ds(g * 16, 16)], jnp.float32)
        for ri in range(16):
            a_s = av[ri]
            r = g * 16 + ri
            for q in range(8):
                off = q * 16
                hb[r, pl.ds(off, 16)] = hb[r, pl.ds(off, 16)] * a_s
        return cc

    lax.fori_loop(0, CH // 16, grp_body, 0)


def _k3_body(rec_hbm, h0_hbm, h1_hbm, zrows_hbm,
             aggp_hbm,
             rec0, rec1, rec2, rec3, hb0, hb1, sg0, sg1, ss0, ss1, agg_s):
    cid = lax.axis_index("c")
    sid = lax.axis_index("s")
    wid = sid * 2 + cid
    tid = sid
    recs = (rec0, rec1, rec2, rec3)
    hbs = (hb0, hb1)
    sgs = (sg0, sg1)
    sss = (ss0, ss1)

    for half in range(2):
        h_hbm = h0_hbm if half == 0 else h1_hbm
        pltpu.sync_copy(zrows_hbm, agg_s.at[pl.ds(tid * RPT, RPT)])
        plsc.subcore_barrier()

        def slot(j, b, r, first, last):
            pltpu.make_async_copy(h_hbm.at[recs[r].at[0]], hbs[b],
                                  sgs[b]).wait()
            if not last:
                pltpu.sync_copy(rec_hbm.at[wid, j + 1], recs[(r + 1) % 4])
            if not first:
                pltpu.make_async_copy(h_hbm.at[recs[(r + 3) % 4].at[0]],
                                      hbs[1 - b], sss[1 - b]).wait()
            if not last:
                pltpu.async_copy(h_hbm.at[recs[(r + 1) % 4].at[0]],
                                 hbs[1 - b], sgs[1 - b])
            _scale_rows(hbs[b], recs[r])
            if last:
                pltpu.sync_copy(hbs[b], agg_s.at[recs[r].at[1]], add=True)
            else:
                pltpu.async_copy(hbs[b], agg_s.at[recs[r].at[1]], sss[b],
                                 add=True)

        pltpu.sync_copy(rec_hbm.at[wid, 0], recs[0])
        pltpu.async_copy(h_hbm.at[recs[0].at[0]], hbs[0], sgs[0])
        slot(0, 0, 0, first=True, last=False)

        def quad_body(t, c):
            jb = 4 * t
            slot(jb + 1, 1, 1, first=False, last=False)
            slot(jb + 2, 0, 2, first=False, last=False)
            slot(jb + 3, 1, 3, first=False, last=False)
            slot(jb + 4, 0, 0, first=False, last=(False))
            return c

        lax.fori_loop(0, (NCH - 5) // 4, quad_body, 0)
        jb = NCH - 5
        slot(jb + 1, 1, 1, first=False, last=False)
        slot(jb + 2, 0, 2, first=False, last=False)
        slot(jb + 3, 1, 3, first=False, last=False)
        slot(jb + 4, 0, 0, first=False, last=True)

        plsc.subcore_barrier()
        pltpu.sync_copy(agg_s.at[pl.ds(tid * RPT, RPT)],
                        aggp_hbm.at[cid, half, pl.ds(tid * RPT, RPT)])
        plsc.subcore_barrier()


def _k3(rec, h0, h1, zrows):
    mesh = plsc.VectorSubcoreMesh(core_axis_name="c", subcore_axis_name="s")
    f = functools.partial(
        pl.kernel,
        mesh=mesh,
        compiler_params=pltpu.CompilerParams(needs_layout_passes=False),
        out_type=jax.ShapeDtypeStruct((2, 2, NP, 128), jnp.float32),
        scratch_types=[
            pltpu.VMEM((3, CH), jnp.int32),
            pltpu.VMEM((3, CH), jnp.int32),
            pltpu.VMEM((3, CH), jnp.int32),
            pltpu.VMEM((3, CH), jnp.int32),
            pltpu.VMEM((CH, 128), jnp.float32),
            pltpu.VMEM((CH, 128), jnp.float32),
            pltpu.SemaphoreType.DMA,
            pltpu.SemaphoreType.DMA,
            pltpu.SemaphoreType.DMA,
            pltpu.SemaphoreType.DMA,
            pltpu.VMEM_SHARED((NP, 128), jnp.float32),
        ],
    )(_k3_body)
    return f(rec, h0, h1, zrows)


def _k3c_body(rec_hbm, ea4_hbm, zrows_hbm, wsump_hbm,
              rec_v, e16, ebuf, ws_s):
    cid = lax.axis_index("c")
    sid = lax.axis_index("s")
    wid = sid * 2 + cid
    tid = sid

    def zb(i, c):
        for q in range(8):
            ebuf[i, pl.ds(q * 16, 16)] = jnp.zeros((16,), jnp.float32)
        return c

    lax.fori_loop(0, CH, zb, 0)
    pltpu.sync_copy(zrows_hbm, ws_s.at[pl.ds(tid * RPT, RPT)])
    plsc.subcore_barrier()

    def chunk_body(j, c):
        pltpu.sync_copy(rec_hbm.at[wid, j], rec_v)
        pltpu.sync_copy(ea4_hbm.at[wid, j], e16)

        def grp_body(g, cc):
            av = plsc.bitcast(rec_v[2, pl.ds(g * 16, 16)], jnp.float32)
            for ri in range(16):
                a_s = av[ri]
                r = g * 16 + ri
                ebuf[r, pl.ds(0, 16)] = e16[r, pl.ds(0, 16)] * a_s
            return cc

        lax.fori_loop(0, CH // 16, grp_body, 0)
        pltpu.sync_copy(ebuf, ws_s.at[rec_v.at[1]], add=True)
        return c

    lax.fori_loop(0, NCH, chunk_body, 0)
    plsc.subcore_barrier()
    pltpu.sync_copy(ws_s.at[pl.ds(tid * RPT, RPT)],
                    wsump_hbm.at[cid, pl.ds(tid * RPT, RPT)])


def _k3c(rec, ea4, zrows):
    mesh = plsc.VectorSubcoreMesh(core_axis_name="c", subcore_axis_name="s")
    f = functools.partial(
        pl.kernel,
        mesh=mesh,
        compiler_params=pltpu.CompilerParams(needs_layout_passes=False),
        out_type=jax.ShapeDtypeStruct((2, NP, 128), jnp.float32),
        scratch_types=[
            pltpu.VMEM((3, CH), jnp.int32),
            pltpu.VMEM((CH, 16), jnp.float32),
            pltpu.VMEM((CH, 128), jnp.float32),
            pltpu.VMEM_SHARED((NP, 128), jnp.float32),
        ],
    )(_k3c_body)
    return f(rec, ea4, zrows)


def _k4a_body(aggp_ref, wsump_ref, h0_ref, h1_ref, den_ref, we_ref, web_ref,
              out_ref):
    aggp = aggp_ref[...]
    agg = jnp.concatenate(
        [aggp[0, 0] + aggp[1, 0], aggp[0, 1] + aggp[1, 1]], axis=-1)
    wsum = (wsump_ref[...][0] + wsump_ref[...][1])[:, :B_IN]
    he = lax.dot_general(wsum, we_ref[...], (((1,), (1,)), ((), ())),
                         preferred_element_type=jnp.float32)
    den = den_ref[...][:, 0]
    asum = den / (den + 1e-16)
    h = jnp.concatenate([h0_ref[...], h1_ref[...]], axis=-1)
    out_ref[...] = jnp.maximum(
        agg + he + asum[:, None] * web_ref[...][None, :] + h, 0.0)


def _k4a(aggp, wsump, h0, h1, denom, conv_We, conv_We_b):
    grid = (N // _RB,)
    return pl.pallas_call(
        _k4a_body,
        grid=grid,
        in_specs=[
            pl.BlockSpec((2, 2, _RB, 128), lambda i: (0, 0, i, 0)),
            pl.BlockSpec((2, _RB, 128), lambda i: (0, i, 0)),
            pl.BlockSpec((_RB, 128), lambda i: (i, 0)),
            pl.BlockSpec((_RB, 128), lambda i: (i, 0)),
            pl.BlockSpec((_RB, 1), lambda i: (i, 0)),
            pl.BlockSpec((DIM, B_IN), lambda i: (0, 0)),
            pl.BlockSpec((DIM,), lambda i: (0,)),
        ],
        out_specs=pl.BlockSpec((_RB, DIM), lambda i: (i, 0)),
        out_shape=jax.ShapeDtypeStruct((N, DIM), jnp.float32),
    )(aggp, wsump, h0, h1, denom.reshape(N, 1), conv_We, conv_We_b)


def _k4b_body(out_ref, batch_ref, fp_ref, fc1W_ref, fc1b_ref, bng_ref,
              bnb_ref, wih_ref, whh_ref, bih_ref, bhh_ref, lin1W_ref,
              lin1b_ref, lin2W_ref, lin2b_ref, res_ref):
    outm = out_ref[...]
    bvec = batch_ref[...]
    gid = lax.broadcasted_iota(jnp.int32, (N, B), 1)
    mask = bvec[:, None] == gid

    q_star = jnp.zeros((B, 2 * DIM), jnp.float32)
    h_l = jnp.zeros((B, DIM), jnp.float32)
    c_l = jnp.zeros((B, DIM), jnp.float32)
    wih = wih_ref[...]
    whh = whh_ref[...]
    bih = bih_ref[...]
    bhh = bhh_ref[...]
    for _ in range(3):
        gates = (lax.dot_general(q_star, wih, (((1,), (1,)), ((), ())),
                                 preferred_element_type=jnp.float32)
                 + bih[None, :]
                 + lax.dot_general(h_l, whh, (((1,), (1,)), ((), ())),
                                   preferred_element_type=jnp.float32)
                 + bhh[None, :])
        i_g = gates[:, :DIM]
        f_g = gates[:, DIM:2 * DIM]
        g_g = gates[:, 2 * DIM:3 * DIM]
        o_g = gates[:, 3 * DIM:]
        c_l = jax.nn.sigmoid(f_g) * c_l + jax.nn.sigmoid(i_g) * jnp.tanh(g_g)
        h_l = jax.nn.sigmoid(o_g) * jnp.tanh(c_l)
        q = h_l
        m = lax.dot_general(outm, q, (((1,), (1,)), ((), ())),
                            preferred_element_type=jnp.float32)
        emax = jnp.max(jnp.where(mask, m, -1e30), axis=0)
        anum = jnp.where(mask, jnp.exp(m - emax[None, :]), 0.0)
        den = jnp.sum(anum, axis=0)
        amat = anum / (den + 1e-16)[None, :]
        r = lax.dot_general(amat, outm, (((0,), (0,)), ((), ())),
                            preferred_element_type=jnp.float32)
        q_star = jnp.concatenate([q, r], axis=-1)

    g_out = jnp.maximum(
        lax.dot_general(q_star, lin1W_ref[...], (((1,), (1,)), ((), ())),
                        preferred_element_type=jnp.float32)
        + lin1b_ref[...][None, :], 0.0)

    h_fp = lax.dot_general(fp_ref[...], fc1W_ref[...], (((1,), (1,)), ((), ())),
                           preferred_element_type=jnp.float32) + fc1b_ref[...][None, :]
    h_fp = h_fp / jnp.sqrt(1.0 + 1e-5) * bng_ref[...][None, :] + bnb_ref[...][None, :]
    out_fp = jnp.where(h_fp > 0, h_fp, jnp.exp(h_fp) - 1.0)

    cat = jnp.concatenate([g_out, out_fp], axis=-1)
    res_ref[...] = (lax.dot_general(cat, lin2W_ref[...], (((1,), (1,)), ((), ())),
                                    preferred_element_type=jnp.float32)
                    + lin2b_ref[...][None, :])


def _k4b(out, batch, fp, fc1_W, fc1_b, bn_g, bn_b, lstm_W_ih, lstm_W_hh,
         lstm_b_ih, lstm_b_hh, lin1_W, lin1_b, lin2_W, lin2_b):
    return pl.pallas_call(
        _k4b_body,
        out_shape=jax.ShapeDtypeStruct((B, OUT), jnp.float32),
    )(out, batch, fp, fc1_W, fc1_b, bn_g, bn_b, lstm_W_ih, lstm_W_hh,
      lstm_b_ih, lstm_b_hh, lin1_W, lin1_b, lin2_W, lin2_b)


def kernel(fp, x, edge_attr, edge_index, batch, fc1_W, fc1_b, bn_g, bn_b,
           lin0_W, lin0_b, conv_W, conv_b, conv_We, conv_We_b,
           a_src, a_dst, a_e, lstm_W_ih, lstm_W_hh, lstm_b_ih, lstm_b_hh,
           lin1_W, lin1_b, lin2_W, lin2_b):
    src = edge_index[0]
    dst = edge_index[1]

    h0, h1, hs2, hd2 = _k0a(x, lin0_W, lin0_b, conv_W, conv_b, a_src, a_dst)
    hs = hs2.reshape(N)
    hd = hd2.reshape(N)
    ea = _k0b(edge_attr, conv_We, a_e, conv_We_b).reshape(E)
    expv, denp = _k1(src, dst, ea, hs, hd)
    denom = _k2(denp.reshape(NW, N))
    alpha = _k2b(dst, expv, denom)

    rec = jnp.stack(
        [src.reshape(NW, NCH, CH),
         dst.reshape(NW, NCH, CH),
         lax.bitcast_convert_type(alpha, jnp.int32).reshape(NW, NCH, CH)],
        axis=2)
    ea4 = jnp.pad(edge_attr, ((0, 0), (0, 16 - B_IN))).reshape(NW, NCH, CH, 16)
    zrows = jnp.zeros((RPT, 128), jnp.float32)

    aggp = _k3(rec, h0, h1, zrows)
    wsump = _k3c(rec, ea4, zrows)
    out = _k4a(aggp, wsump, h0, h1, denom, conv_We, conv_We_b)
    return _k4b(out, batch, fp, fc1_W, fc1_b, bn_g, bn_b, lstm_W_ih,
                lstm_W_hh, lstm_b_ih, lstm_b_hh, lin1_W, lin1_b, lin2_W,
                lin2_b)


# speedup vs baseline: 8.6839x; 1.1128x over previous
import functools

import jax
import jax.numpy as jnp
from jax import lax
from jax.experimental import pallas as pl
from jax.experimental.pallas import tpu as pltpu
from jax.experimental.pallas import tpu_sc as plsc

N = 10000
E = 320000
B = 128
DIM = 256
MOL_IN = 15
B_IN = 14
FP_DIM = 1024
FP_LIN = 64
OUT = 2

NW = 32
EPW = E // NW
CH = 80
NCH = EPW // CH
NP = 10240
RPT = NP // 16


_RB = 2000


def _k0a_body(x_ref, w0_ref, b0_ref, wc_ref, bc_ref, asrc_ref, adst_ref,
              h0_ref, h1_ref, hs_ref, hd_ref):
    x = x_ref[...]
    out0 = lax.dot_general(x, w0_ref[...], (((1,), (1,)), ((), ())),
                           preferred_element_type=jnp.float32)
    out0 = jnp.maximum(out0 + b0_ref[...][None, :], 0.0)
    h = lax.dot_general(out0, wc_ref[...], (((1,), (1,)), ((), ())),
                        preferred_element_type=jnp.float32)
    h = h + bc_ref[...][None, :]
    h0_ref[...] = h[:, :128]
    h1_ref[...] = h[:, 128:]
    hs_ref[...] = jnp.sum(h * asrc_ref[...][None, :], axis=1)[:, None]
    hd_ref[...] = jnp.sum(h * adst_ref[...][None, :], axis=1)[:, None]


def _k0a(x, lin0_W, lin0_b, conv_W, conv_b, a_src, a_dst):
    grid = (N // _RB,)
    return pl.pallas_call(
        _k0a_body,
        grid=grid,
        in_specs=[
            pl.BlockSpec((_RB, MOL_IN), lambda i: (i, 0)),
            pl.BlockSpec((DIM, MOL_IN), lambda i: (0, 0)),
            pl.BlockSpec((DIM,), lambda i: (0,)),
            pl.BlockSpec((DIM, DIM), lambda i: (0, 0)),
            pl.BlockSpec((DIM,), lambda i: (0,)),
            pl.BlockSpec((DIM,), lambda i: (0,)),
            pl.BlockSpec((DIM,), lambda i: (0,)),
        ],
        out_specs=[
            pl.BlockSpec((_RB, 128), lambda i: (i, 0)),
            pl.BlockSpec((_RB, 128), lambda i: (i, 0)),
            pl.BlockSpec((_RB, 1), lambda i: (i, 0)),
            pl.BlockSpec((_RB, 1), lambda i: (i, 0)),
        ],
        out_shape=[
            jax.ShapeDtypeStruct((N, 128), jnp.float32),
            jax.ShapeDtypeStruct((N, 128), jnp.float32),
            jax.ShapeDtypeStruct((N, 1), jnp.float32),
            jax.ShapeDtypeStruct((N, 1), jnp.float32),
        ],
    )(x, lin0_W, lin0_b, conv_W, conv_b, a_src, a_dst)


_EB = 16000


def _k0b_body(eattr_ref, we_ref, ae_ref, web_ref, ea_ref):
    ae = ae_ref[...]
    v = jnp.sum(we_ref[...] * ae[:, None], axis=0)
    c = jnp.sum(ae * web_ref[...])
    ea_ref[...] = (jnp.sum(eattr_ref[...] * v[None, :], axis=1) + c)[:, None]


def _k0b(edge_attr, conv_We, a_e, conv_We_b):
    grid = (E // _EB,)
    return pl.pallas_call(
        _k0b_body,
        grid=grid,
        in_specs=[
            pl.BlockSpec((_EB, B_IN), lambda i: (i, 0)),
            pl.BlockSpec((DIM, B_IN), lambda i: (0, 0)),
            pl.BlockSpec((DIM,), lambda i: (0,)),
            pl.BlockSpec((DIM,), lambda i: (0,)),
        ],
        out_specs=pl.BlockSpec((_EB, 1), lambda i: (i, 0)),
        out_shape=jax.ShapeDtypeStruct((E, 1), jnp.float32),
    )(edge_attr, conv_We, a_e, conv_We_b)


def _k1_body(src_hbm, dst_hbm, ea_hbm, hs_hbm, hd_hbm,
             expv_hbm, denp_hbm,
             src_v, dst_v, ea_v, hs_v, hd_v, ex_v, den_v):
    cid = lax.axis_index("c")
    sid = lax.axis_index("s")
    wid = sid * 2 + cid
    base = wid * EPW
    pltpu.sync_copy(src_hbm.at[pl.ds(base, EPW)], src_v)
    pltpu.sync_copy(dst_hbm.at[pl.ds(base, EPW)], dst_v)
    pltpu.sync_copy(ea_hbm.at[pl.ds(base, EPW)], ea_v)
    pltpu.sync_copy(hs_hbm, hs_v)
    pltpu.sync_copy(hd_hbm, hd_v)

    def zero_body(i, c):
        den_v[pl.ds(i * 16, 16)] = jnp.zeros((16,), jnp.float32)
        return c

    lax.fori_loop(0, N // 16, zero_body, 0)

    def body(e, c):
        off = e * 16
        s16 = src_v[pl.ds(off, 16)]
        d16 = dst_v[pl.ds(off, 16)]
        a16 = ea_v[pl.ds(off, 16)]
        hsg = plsc.load_gather(hs_v, [s16])
        hdg = plsc.load_gather(hd_v, [d16])
        t = hsg + hdg + a16
        lg = jnp.maximum(t, t * 0.2)
        ex = jnp.exp(lg)
        ex_v[pl.ds(off, 16)] = ex
        plsc.addupdate_scatter(den_v, [d16], ex)
        return c

    lax.fori_loop(0, EPW // 16, body, 0)
    pltpu.sync_copy(ex_v, expv_hbm.at[pl.ds(base, EPW)])
    pltpu.sync_copy(den_v, denp_hbm.at[wid, 0])


def _k1(src, dst, ea, hs, hd):
    mesh = plsc.VectorSubcoreMesh(core_axis_name="c", subcore_axis_name="s")
    f = functools.partial(
        pl.kernel,
        mesh=mesh,
        compiler_params=pltpu.CompilerParams(needs_layout_passes=False),
        out_type=[
            jax.ShapeDtypeStruct((E,), jnp.float32),
            jax.ShapeDtypeStruct((NW, 1, N), jnp.float32),
        ],
        scratch_types=[
            pltpu.VMEM((EPW,), jnp.int32),
            pltpu.VMEM((EPW,), jnp.int32),
            pltpu.VMEM((EPW,), jnp.float32),
            pltpu.VMEM((N,), jnp.float32),
            pltpu.VMEM((N,), jnp.float32),
            pltpu.VMEM((EPW,), jnp.float32),
            pltpu.VMEM((N,), jnp.float32),
        ],
    )(_k1_body)
    return f(src, dst, ea, hs, hd)


def _k2_body(denp_ref, den_ref):
    den_ref[...] = jnp.sum(denp_ref[...], axis=0)


def _k2(denp):
    return pl.pallas_call(
        _k2_body,
        out_shape=jax.ShapeDtypeStruct((N,), jnp.float32),
    )(denp)


def _k2b_body(dst_hbm, ex_hbm, den_hbm, al_hbm, dst_v, ex_v, den_v, al_v):
    cid = lax.axis_index("c")
    sid = lax.axis_index("s")
    wid = sid * 2 + cid
    base = wid * EPW
    pltpu.sync_copy(dst_hbm.at[pl.ds(base, EPW)], dst_v)
    pltpu.sync_copy(ex_hbm.at[pl.ds(base, EPW)], ex_v)
    pltpu.sync_copy(den_hbm, den_v)

    def body(e, c):
        off = e * 16
        d16 = dst_v[pl.ds(off, 16)]
        deng = plsc.load_gather(den_v, [d16])
        al_v[pl.ds(off, 16)] = ex_v[pl.ds(off, 16)] / (deng + 1e-16)
        return c

    lax.fori_loop(0, EPW // 16, body, 0)
    pltpu.sync_copy(al_v, al_hbm.at[pl.ds(base, EPW)])


def _k2b(dst, expv, denom):
    mesh = plsc.VectorSubcoreMesh(core_axis_name="c", subcore_axis_name="s")
    f = functools.partial(
        pl.kernel,
        mesh=mesh,
        compiler_params=pltpu.CompilerParams(needs_layout_passes=False),
        out_type=jax.ShapeDtypeStruct((E,), jnp.float32),
        scratch_types=[
            pltpu.VMEM((EPW,), jnp.int32),
            pltpu.VMEM((EPW,), jnp.float32),
            pltpu.VMEM((N,), jnp.float32),
            pltpu.VMEM((EPW,), jnp.float32),
        ],
    )(_k2b_body)
    return f(dst, expv, denom)


def _scale_rows(hb, rec):
    def grp_body(g, cc):
        av = plsc.bitcast(rec[2, pl.ds(g * 16, 16)], jnp.float32)
        for ri in range(16):
            a_s = av[ri]
            r = g * 16 + ri
            for q in range(8):
                off = q * 16
                hb[r, pl.ds(off, 16)] = hb[r, pl.ds(off, 16)] * a_s
        return cc

    lax.fori_loop(0, CH // 16, grp_body, 0)


def _k3_body(rec_hbm, h0_hbm, h1_hbm, zrows_hbm,
             aggp_hbm,
             rec0, rec1, rec2, rec3, hb0, hb1, sg0, sg1, ss0, ss1, agg_s):
    cid = lax.axis_index("c")
    sid = lax.axis_index("s")
    wid = sid * 2 + cid
    tid = sid
    recs = (rec0, rec1, rec2, rec3)
    hbs = (hb0, hb1)
    sgs = (sg0, sg1)
    sss = (ss0, ss1)

    for half in range(2):
        h_hbm = h0_hbm if half == 0 else h1_hbm
        pltpu.sync_copy(zrows_hbm, agg_s.at[pl.ds(tid * RPT, RPT)])
        plsc.subcore_barrier()

        def slot(j, b, r, first, last):
            pltpu.make_async_copy(h_hbm.at[recs[r].at[0]], hbs[b],
                                  sgs[b]).wait()
            if not last:
                pltpu.sync_copy(rec_hbm.at[wid, j + 1], recs[(r + 1) % 4])
            if not first:
                pltpu.make_async_copy(h_hbm.at[recs[(r + 3) % 4].at[0]],
                                      hbs[1 - b], sss[1 - b]).wait()
            if not last:
                pltpu.async_copy(h_hbm.at[recs[(r + 1) % 4].at[0]],
                                 hbs[1 - b], sgs[1 - b])
            _scale_rows(hbs[b], recs[r])
            if last:
                pltpu.sync_copy(hbs[b], agg_s.at[recs[r].at[1]], add=True)
            else:
                pltpu.async_copy(hbs[b], agg_s.at[recs[r].at[1]], sss[b],
                                 add=True)

        pltpu.sync_copy(rec_hbm.at[wid, 0], recs[0])
        pltpu.async_copy(h_hbm.at[recs[0].at[0]], hbs[0], sgs[0])
        slot(0, 0, 0, first=True, last=False)

        def quad_body(t, c):
            jb = 4 * t
            slot(jb + 1, 1, 1, first=False, last=False)
            slot(jb + 2, 0, 2, first=False, last=False)
            slot(jb + 3, 1, 3, first=False, last=False)
            slot(jb + 4, 0, 0, first=False, last=(False))
            return c

        lax.fori_loop(0, (NCH - 5) // 4, quad_body, 0)
        jb = NCH - 5
        slot(jb + 1, 1, 1, first=False, last=False)
        slot(jb + 2, 0, 2, first=False, last=False)
        slot(jb + 3, 1, 3, first=False, last=False)
        slot(jb + 4, 0, 0, first=False, last=True)

        plsc.subcore_barrier()
        pltpu.sync_copy(agg_s.at[pl.ds(tid * RPT, RPT)],
                        aggp_hbm.at[cid, half, pl.ds(tid * RPT, RPT)])
        plsc.subcore_barrier()


def _k3(rec, h0, h1, zrows):
    mesh = plsc.VectorSubcoreMesh(core_axis_name="c", subcore_axis_name="s")
    f = functools.partial(
        pl.kernel,
        mesh=mesh,
        compiler_params=pltpu.CompilerParams(needs_layout_passes=False),
        out_type=jax.ShapeDtypeStruct((2, 2, NP, 128), jnp.float32),
        scratch_types=[
            pltpu.VMEM((3, CH), jnp.int32),
            pltpu.VMEM((3, CH), jnp.int32),
            pltpu.VMEM((3, CH), jnp.int32),
            pltpu.VMEM((3, CH), jnp.int32),
            pltpu.VMEM((CH, 128), jnp.float32),
            pltpu.VMEM((CH, 128), jnp.float32),
            pltpu.SemaphoreType.DMA,
            pltpu.SemaphoreType.DMA,
            pltpu.SemaphoreType.DMA,
            pltpu.SemaphoreType.DMA,
            pltpu.VMEM_SHARED((NP, 128), jnp.float32),
        ],
    )(_k3_body)
    return f(rec, h0, h1, zrows)


def _k3c_body(rec_hbm, ea4_hbm, zrows_hbm, wsump_hbm,
              rec0, rec1, rec2, rec3, ea_0, ea_1, eb0, eb1,
              sr0, sr1, se0, se1, ss0, ss1, ws_s):
    cid = lax.axis_index("c")
    sid = lax.axis_index("s")
    wid = sid * 2 + cid
    tid = sid
    recs = (rec0, rec1, rec2, rec3)
    eas = (ea_0, ea_1)
    ebs = (eb0, eb1)
    srs = (sr0, sr1)
    ses = (se0, se1)
    sss = (ss0, ss1)

    def zb(i, c):
        for q in range(8):
            eb0[i, pl.ds(q * 16, 16)] = jnp.zeros((16,), jnp.float32)
            eb1[i, pl.ds(q * 16, 16)] = jnp.zeros((16,), jnp.float32)
        return c

    lax.fori_loop(0, CH, zb, 0)
    pltpu.sync_copy(zrows_hbm, ws_s.at[pl.ds(tid * RPT, RPT)])
    plsc.subcore_barrier()

    def slot(j, b, r, first, second, last):
        if not first:
            pltpu.make_async_copy(rec_hbm.at[wid, j], recs[r], srs[b]).wait()
            pltpu.make_async_copy(ea4_hbm.at[wid, j], eas[b], ses[b]).wait()
        if not last:
            pltpu.async_copy(rec_hbm.at[wid, j + 1], recs[(r + 1) % 4],
                             srs[1 - b])
            pltpu.async_copy(ea4_hbm.at[wid, j + 1], eas[1 - b], ses[1 - b])
        if not (first or second):
            pltpu.make_async_copy(ebs[b], ws_s.at[recs[(r + 2) % 4].at[1]],
                                  sss[b]).wait()

        def grp_body(g, cc):
            av = plsc.bitcast(recs[r][2, pl.ds(g * 16, 16)], jnp.float32)
            for ri in range(16):
                a_s = av[ri]
                rr = g * 16 + ri
                ebs[b][rr, pl.ds(0, 16)] = eas[b][rr, pl.ds(0, 16)] * a_s
            return cc

        lax.fori_loop(0, CH // 16, grp_body, 0)
        if last:
            pltpu.sync_copy(ebs[b], ws_s.at[recs[r].at[1]], add=True)
        else:
            pltpu.async_copy(ebs[b], ws_s.at[recs[r].at[1]], sss[b], add=True)

    pltpu.sync_copy(rec_hbm.at[wid, 0], recs[0])
    pltpu.sync_copy(ea4_hbm.at[wid, 0], eas[0])
    slot(0, 0, 0, first=True, second=False, last=False)
    slot(1, 1, 1, first=False, second=True, last=False)

    def quad_body(t, c):
        jb = 4 * t
        slot(jb + 2, 0, 2, first=False, second=False, last=False)
        slot(jb + 3, 1, 3, first=False, second=False, last=False)
        slot(jb + 4, 0, 0, first=False, second=False, last=False)
        slot(jb + 5, 1, 1, first=False, second=False, last=False)
        return c

    lax.fori_loop(0, (NCH - 5) // 4, quad_body, 0)
    slot(NCH - 3, 0, 2, first=False, second=False, last=False)
    slot(NCH - 2, 1, 3, first=False, second=False, last=False)
    slot(NCH - 1, 0, 0, first=False, second=False, last=True)
    pltpu.make_async_copy(ebs[1], ws_s.at[recs[3].at[1]], sss[1]).wait()

    plsc.subcore_barrier()
    pltpu.sync_copy(ws_s.at[pl.ds(tid * RPT, RPT)],
                    wsump_hbm.at[cid, pl.ds(tid * RPT, RPT)])


def _k3c(rec, ea4, zrows):
    mesh = plsc.VectorSubcoreMesh(core_axis_name="c", subcore_axis_name="s")
    f = functools.partial(
        pl.kernel,
        mesh=mesh,
        compiler_params=pltpu.CompilerParams(needs_layout_passes=False),
        out_type=jax.ShapeDtypeStruct((2, NP, 128), jnp.float32),
        scratch_types=[
            pltpu.VMEM((3, CH), jnp.int32),
            pltpu.VMEM((3, CH), jnp.int32),
            pltpu.VMEM((3, CH), jnp.int32),
            pltpu.VMEM((3, CH), jnp.int32),
            pltpu.VMEM((CH, 16), jnp.float32),
            pltpu.VMEM((CH, 16), jnp.float32),
            pltpu.VMEM((CH, 128), jnp.float32),
            pltpu.VMEM((CH, 128), jnp.float32),
            pltpu.SemaphoreType.DMA,
            pltpu.SemaphoreType.DMA,
            pltpu.SemaphoreType.DMA,
            pltpu.SemaphoreType.DMA,
            pltpu.SemaphoreType.DMA,
            pltpu.SemaphoreType.DMA,
            pltpu.VMEM_SHARED((NP, 128), jnp.float32),
        ],
    )(_k3c_body)
    return f(rec, ea4, zrows)


def _k4a_body(aggp_ref, wsump_ref, h0_ref, h1_ref, den_ref, we_ref, web_ref,
              out_ref):
    aggp = aggp_ref[...]
    agg = jnp.concatenate(
        [aggp[0, 0] + aggp[1, 0], aggp[0, 1] + aggp[1, 1]], axis=-1)
    wsum = (wsump_ref[...][0] + wsump_ref[...][1])[:, :B_IN]
    he = lax.dot_general(wsum, we_ref[...], (((1,), (1,)), ((), ())),
                         preferred_element_type=jnp.float32)
    den = den_ref[...][:, 0]
    asum = den / (den + 1e-16)
    h = jnp.concatenate([h0_ref[...], h1_ref[...]], axis=-1)
    out_ref[...] = jnp.maximum(
        agg + he + asum[:, None] * web_ref[...][None, :] + h, 0.0)


def _k4a(aggp, wsump, h0, h1, denom, conv_We, conv_We_b):
    grid = (N // _RB,)
    return pl.pallas_call(
        _k4a_body,
        grid=grid,
        in_specs=[
            pl.BlockSpec((2, 2, _RB, 128), lambda i: (0, 0, i, 0)),
            pl.BlockSpec((2, _RB, 128), lambda i: (0, i, 0)),
            pl.BlockSpec((_RB, 128), lambda i: (i, 0)),
            pl.BlockSpec((_RB, 128), lambda i: (i, 0)),
            pl.BlockSpec((_RB, 1), lambda i: (i, 0)),
            pl.BlockSpec((DIM, B_IN), lambda i: (0, 0)),
            pl.BlockSpec((DIM,), lambda i: (0,)),
        ],
        out_specs=pl.BlockSpec((_RB, DIM), lambda i: (i, 0)),
        out_shape=jax.ShapeDtypeStruct((N, DIM), jnp.float32),
    )(aggp, wsump, h0, h1, denom.reshape(N, 1), conv_We, conv_We_b)


def _k4b_body(out_ref, batch_ref, fp_ref, fc1W_ref, fc1b_ref, bng_ref,
              bnb_ref, wih_ref, whh_ref, bih_ref, bhh_ref, lin1W_ref,
              lin1b_ref, lin2W_ref, lin2b_ref, res_ref):
    outm = out_ref[...]
    bvec = batch_ref[...]
    gid = lax.broadcasted_iota(jnp.int32, (N, B), 1)
    mask = bvec[:, None] == gid

    q_star = jnp.zeros((B, 2 * DIM), jnp.float32)
    h_l = jnp.zeros((B, DIM), jnp.float32)
    c_l = jnp.zeros((B, DIM), jnp.float32)
    wih = wih_ref[...]
    whh = whh_ref[...]
    bih = bih_ref[...]
    bhh = bhh_ref[...]
    for _ in range(3):
        gates = (lax.dot_general(q_star, wih, (((1,), (1,)), ((), ())),
                                 preferred_element_type=jnp.float32)
                 + bih[None, :]
                 + lax.dot_general(h_l, whh, (((1,), (1,)), ((), ())),
                                   preferred_element_type=jnp.float32)
                 + bhh[None, :])
        i_g = gates[:, :DIM]
        f_g = gates[:, DIM:2 * DIM]
        g_g = gates[:, 2 * DIM:3 * DIM]
        o_g = gates[:, 3 * DIM:]
        c_l = jax.nn.sigmoid(f_g) * c_l + jax.nn.sigmoid(i_g) * jnp.tanh(g_g)
        h_l = jax.nn.sigmoid(o_g) * jnp.tanh(c_l)
        q = h_l
        m = lax.dot_general(outm, q, (((1,), (1,)), ((), ())),
                            preferred_element_type=jnp.float32)
        emax = jnp.max(jnp.where(mask, m, -1e30), axis=0)
        anum = jnp.where(mask, jnp.exp(m - emax[None, :]), 0.0)
        den = jnp.sum(anum, axis=0)
        amat = anum / (den + 1e-16)[None, :]
        r = lax.dot_general(amat, outm, (((0,), (0,)), ((), ())),
                            preferred_element_type=jnp.float32)
        q_star = jnp.concatenate([q, r], axis=-1)

    g_out = jnp.maximum(
        lax.dot_general(q_star, lin1W_ref[...], (((1,), (1,)), ((), ())),
                        preferred_element_type=jnp.float32)
        + lin1b_ref[...][None, :], 0.0)

    h_fp = lax.dot_general(fp_ref[...], fc1W_ref[...], (((1,), (1,)), ((), ())),
                           preferred_element_type=jnp.float32) + fc1b_ref[...][None, :]
    h_fp = h_fp / jnp.sqrt(1.0 + 1e-5) * bng_ref[...][None, :] + bnb_ref[...][None, :]
    out_fp = jnp.where(h_fp > 0, h_fp, jnp.exp(h_fp) - 1.0)

    cat = jnp.concatenate([g_out, out_fp], axis=-1)
    res_ref[...] = (lax.dot_general(cat, lin2W_ref[...], (((1,), (1,)), ((), ())),
                                    preferred_element_type=jnp.float32)
                    + lin2b_ref[...][None, :])


def _k4b(out, batch, fp, fc1_W, fc1_b, bn_g, bn_b, lstm_W_ih, lstm_W_hh,
         lstm_b_ih, lstm_b_hh, lin1_W, lin1_b, lin2_W, lin2_b):
    return pl.pallas_call(
        _k4b_body,
        out_shape=jax.ShapeDtypeStruct((B, OUT), jnp.float32),
    )(out, batch, fp, fc1_W, fc1_b, bn_g, bn_b, lstm_W_ih, lstm_W_hh,
      lstm_b_ih, lstm_b_hh, lin1_W, lin1_b, lin2_W, lin2_b)


def kernel(fp, x, edge_attr, edge_index, batch, fc1_W, fc1_b, bn_g, bn_b,
           lin0_W, lin0_b, conv_W, conv_b, conv_We, conv_We_b,
           a_src, a_dst, a_e, lstm_W_ih, lstm_W_hh, lstm_b_ih, lstm_b_hh,
           lin1_W, lin1_b, lin2_W, lin2_b):
    src = edge_index[0]
    dst = edge_index[1]

    h0, h1, hs2, hd2 = _k0a(x, lin0_W, lin0_b, conv_W, conv_b, a_src, a_dst)
    hs = hs2.reshape(N)
    hd = hd2.reshape(N)
    ea = _k0b(edge_attr, conv_We, a_e, conv_We_b).reshape(E)
    expv, denp = _k1(src, dst, ea, hs, hd)
    denom = _k2(denp.reshape(NW, N))
    alpha = _k2b(dst, expv, denom)

    rec = jnp.stack(
        [src.reshape(NW, NCH, CH),
         dst.reshape(NW, NCH, CH),
         lax.bitcast_convert_type(alpha, jnp.int32).reshape(NW, NCH, CH)],
        axis=2)
    ea4 = jnp.pad(edge_attr, ((0, 0), (0, 16 - B_IN))).reshape(NW, NCH, CH, 16)
    zrows = jnp.zeros((RPT, 128), jnp.float32)

    aggp = _k3(rec, h0, h1, zrows)
    wsump = _k3c(rec, ea4, zrows)
    out = _k4a(aggp, wsump, h0, h1, denom, conv_We, conv_We_b)
    return _k4b(out, batch, fp, fc1_W, fc1_b, bn_g, bn_b, lstm_W_ih,
                lstm_W_hh, lstm_b_ih, lstm_b_hh, lin1_W, lin1_b, lin2_W,
                lin2_b)


# speedup vs baseline: 9.7829x; 1.1266x over previous
import functools

import jax
import jax.numpy as jnp
from jax import lax
from jax.experimental import pallas as pl
from jax.experimental.pallas import tpu as pltpu
from jax.experimental.pallas import tpu_sc as plsc

N = 10000
E = 320000
B = 128
DIM = 256
MOL_IN = 15
B_IN = 14
FP_DIM = 1024
FP_LIN = 64
OUT = 2

NW = 32
EPW = E // NW
CH = 80
NCH = EPW // CH
NP = 10240
RPT = NP // 16


_RB = 2000


def _k0a_body(x_ref, w0_ref, b0_ref, wc_ref, bc_ref, asrc_ref, adst_ref,
              h0_ref, h1_ref, hs_ref, hd_ref):
    x = x_ref[...]
    out0 = lax.dot_general(x, w0_ref[...], (((1,), (1,)), ((), ())),
                           preferred_element_type=jnp.float32)
    out0 = jnp.maximum(out0 + b0_ref[...][None, :], 0.0)
    h = lax.dot_general(out0, wc_ref[...], (((1,), (1,)), ((), ())),
                        preferred_element_type=jnp.float32)
    h = h + bc_ref[...][None, :]
    h0_ref[...] = h[:, :128]
    h1_ref[...] = h[:, 128:]
    hs_ref[...] = jnp.sum(h * asrc_ref[...][None, :], axis=1)[:, None]
    hd_ref[...] = jnp.sum(h * adst_ref[...][None, :], axis=1)[:, None]


def _k0a(x, lin0_W, lin0_b, conv_W, conv_b, a_src, a_dst):
    grid = (N // _RB,)
    return pl.pallas_call(
        _k0a_body,
        grid=grid,
        in_specs=[
            pl.BlockSpec((_RB, MOL_IN), lambda i: (i, 0)),
            pl.BlockSpec((DIM, MOL_IN), lambda i: (0, 0)),
            pl.BlockSpec((DIM,), lambda i: (0,)),
            pl.BlockSpec((DIM, DIM), lambda i: (0, 0)),
            pl.BlockSpec((DIM,), lambda i: (0,)),
            pl.BlockSpec((DIM,), lambda i: (0,)),
            pl.BlockSpec((DIM,), lambda i: (0,)),
        ],
        out_specs=[
            pl.BlockSpec((_RB, 128), lambda i: (i, 0)),
            pl.BlockSpec((_RB, 128), lambda i: (i, 0)),
            pl.BlockSpec((_RB, 1), lambda i: (i, 0)),
            pl.BlockSpec((_RB, 1), lambda i: (i, 0)),
        ],
        out_shape=[
            jax.ShapeDtypeStruct((N, 128), jnp.float32),
            jax.ShapeDtypeStruct((N, 128), jnp.float32),
            jax.ShapeDtypeStruct((N, 1), jnp.float32),
            jax.ShapeDtypeStruct((N, 1), jnp.float32),
        ],
    )(x, lin0_W, lin0_b, conv_W, conv_b, a_src, a_dst)


_EB = 16000


def _k0b_body(eattr_ref, we_ref, ae_ref, web_ref, ea_ref):
    ae = ae_ref[...]
    v = jnp.sum(we_ref[...] * ae[:, None], axis=0)
    c = jnp.sum(ae * web_ref[...])
    ea_ref[...] = (jnp.sum(eattr_ref[...] * v[None, :], axis=1) + c)[:, None]


def _k0b(edge_attr, conv_We, a_e, conv_We_b):
    grid = (E // _EB,)
    return pl.pallas_call(
        _k0b_body,
        grid=grid,
        in_specs=[
            pl.BlockSpec((_EB, B_IN), lambda i: (i, 0)),
            pl.BlockSpec((DIM, B_IN), lambda i: (0, 0)),
            pl.BlockSpec((DIM,), lambda i: (0,)),
            pl.BlockSpec((DIM,), lambda i: (0,)),
        ],
        out_specs=pl.BlockSpec((_EB, 1), lambda i: (i, 0)),
        out_shape=jax.ShapeDtypeStruct((E, 1), jnp.float32),
    )(edge_attr, conv_We, a_e, conv_We_b)


def _k1_body(src_hbm, dst_hbm, ea_hbm, hs_hbm, hd_hbm,
             expv_hbm, denp_hbm,
             src_v, dst_v, ea_v, hs_v, hd_v, ex_v, den_v):
    cid = lax.axis_index("c")
    sid = lax.axis_index("s")
    wid = sid * 2 + cid
    base = wid * EPW
    pltpu.sync_copy(src_hbm.at[pl.ds(base, EPW)], src_v)
    pltpu.sync_copy(dst_hbm.at[pl.ds(base, EPW)], dst_v)
    pltpu.sync_copy(ea_hbm.at[pl.ds(base, EPW)], ea_v)
    pltpu.sync_copy(hs_hbm, hs_v)
    pltpu.sync_copy(hd_hbm, hd_v)

    def zero_body(i, c):
        den_v[pl.ds(i * 16, 16)] = jnp.zeros((16,), jnp.float32)
        return c

    lax.fori_loop(0, N // 16, zero_body, 0)

    def body(e, c):
        off = e * 16
        s16 = src_v[pl.ds(off, 16)]
        d16 = dst_v[pl.ds(off, 16)]
        a16 = ea_v[pl.ds(off, 16)]
        hsg = plsc.load_gather(hs_v, [s16])
        hdg = plsc.load_gather(hd_v, [d16])
        t = hsg + hdg + a16
        lg = jnp.maximum(t, t * 0.2)
        ex = jnp.exp(lg)
        ex_v[pl.ds(off, 16)] = ex
        plsc.addupdate_scatter(den_v, [d16], ex)
        return c

    lax.fori_loop(0, EPW // 16, body, 0)
    pltpu.sync_copy(ex_v, expv_hbm.at[pl.ds(base, EPW)])
    pltpu.sync_copy(den_v, denp_hbm.at[wid, 0])


def _k1(src, dst, ea, hs, hd):
    mesh = plsc.VectorSubcoreMesh(core_axis_name="c", subcore_axis_name="s")
    f = functools.partial(
        pl.kernel,
        mesh=mesh,
        compiler_params=pltpu.CompilerParams(needs_layout_passes=False),
        out_type=[
            jax.ShapeDtypeStruct((E,), jnp.float32),
            jax.ShapeDtypeStruct((NW, 1, N), jnp.float32),
        ],
        scratch_types=[
            pltpu.VMEM((EPW,), jnp.int32),
            pltpu.VMEM((EPW,), jnp.int32),
            pltpu.VMEM((EPW,), jnp.float32),
            pltpu.VMEM((N,), jnp.float32),
            pltpu.VMEM((N,), jnp.float32),
            pltpu.VMEM((EPW,), jnp.float32),
            pltpu.VMEM((N,), jnp.float32),
        ],
    )(_k1_body)
    return f(src, dst, ea, hs, hd)


def _k2_body(denp_ref, den_ref):
    den_ref[...] = jnp.sum(denp_ref[...], axis=0)


def _k2(denp):
    return pl.pallas_call(
        _k2_body,
        out_shape=jax.ShapeDtypeStruct((N,), jnp.float32),
    )(denp)


def _k2b_body(dst_hbm, ex_hbm, den_hbm, al_hbm, dst_v, ex_v, den_v, al_v):
    cid = lax.axis_index("c")
    sid = lax.axis_index("s")
    wid = sid * 2 + cid
    base = wid * EPW
    pltpu.sync_copy(dst_hbm.at[pl.ds(base, EPW)], dst_v)
    pltpu.sync_copy(ex_hbm.at[pl.ds(base, EPW)], ex_v)
    pltpu.sync_copy(den_hbm, den_v)

    def body(e, c):
        off = e * 16
        d16 = dst_v[pl.ds(off, 16)]
        deng = plsc.load_gather(den_v, [d16])
        al_v[pl.ds(off, 16)] = ex_v[pl.ds(off, 16)] / (deng + 1e-16)
        return c

    lax.fori_loop(0, EPW // 16, body, 0)
    pltpu.sync_copy(al_v, al_hbm.at[pl.ds(base, EPW)])


def _k2b(dst, expv, denom):
    mesh = plsc.VectorSubcoreMesh(core_axis_name="c", subcore_axis_name="s")
    f = functools.partial(
        pl.kernel,
        mesh=mesh,
        compiler_params=pltpu.CompilerParams(needs_layout_passes=False),
        out_type=jax.ShapeDtypeStruct((E,), jnp.float32),
        scratch_types=[
            pltpu.VMEM((EPW,), jnp.int32),
            pltpu.VMEM((EPW,), jnp.float32),
            pltpu.VMEM((N,), jnp.float32),
            pltpu.VMEM((EPW,), jnp.float32),
        ],
    )(_k2b_body)
    return f(dst, expv, denom)


def _scale_rows(hb, rec):
    def grp_body(g, cc):
        av = plsc.bitcast(rec[2, pl.ds(g * 16, 16)], jnp.float32)
        for ri in range(16):
            a_s = av[ri]
            r = g * 16 + ri
            for q in range(8):
                off = q * 16
                hb[r, pl.ds(off, 16)] = hb[r, pl.ds(off, 16)] * a_s
        return cc

    lax.fori_loop(0, CH // 16, grp_body, 0)


def _k3_body(rec_hbm, h0_hbm, h1_hbm, zrows_hbm,
             aggp_hbm,
             rec0, rec1, rec2, rec3, hb0, hb1, sg0, sg1, ss0, ss1,
             sr0, sr1, agg_s):
    cid = lax.axis_index("c")
    sid = lax.axis_index("s")
    wid = sid * 2 + cid
    tid = sid
    recs = (rec0, rec1, rec2, rec3)
    hbs = (hb0, hb1)
    sgs = (sg0, sg1)
    sss = (ss0, ss1)
    srs = (sr0, sr1)

    for half in range(2):
        h_hbm = h0_hbm if half == 0 else h1_hbm
        pltpu.sync_copy(zrows_hbm, agg_s.at[pl.ds(tid * RPT, RPT)])
        plsc.subcore_barrier()

        def slot(j, b, r, first, last, prefetch=True):
            pltpu.make_async_copy(h_hbm.at[recs[r].at[0]], hbs[b],
                                  sgs[b]).wait()
            if prefetch:
                pltpu.async_copy(rec_hbm.at[wid, j + 2], recs[(r + 2) % 4],
                                 srs[b])
            if not (first or last):
                pltpu.make_async_copy(rec_hbm.at[wid, j + 1],
                                      recs[(r + 1) % 4], srs[1 - b]).wait()
            if not first:
                pltpu.make_async_copy(h_hbm.at[recs[(r + 3) % 4].at[0]],
                                      hbs[1 - b], sss[1 - b]).wait()
            if not last:
                pltpu.async_copy(h_hbm.at[recs[(r + 1) % 4].at[0]],
                                 hbs[1 - b], sgs[1 - b])
            _scale_rows(hbs[b], recs[r])
            if last:
                pltpu.sync_copy(hbs[b], agg_s.at[recs[r].at[1]], add=True)
            else:
                pltpu.async_copy(hbs[b], agg_s.at[recs[r].at[1]], sss[b],
                                 add=True)

        pltpu.sync_copy(rec_hbm.at[wid, 0], recs[0])
        pltpu.sync_copy(rec_hbm.at[wid, 1], recs[1])
        pltpu.async_copy(h_hbm.at[recs[0].at[0]], hbs[0], sgs[0])
        slot(0, 0, 0, first=True, last=False)

        def quad_body(t, c):
            jb = 4 * t
            slot(jb + 1, 1, 1, first=False, last=False)
            slot(jb + 2, 0, 2, first=False, last=False)
            slot(jb + 3, 1, 3, first=False, last=False)
            slot(jb + 4, 0, 0, first=False, last=False)
            return c

        lax.fori_loop(0, (NCH - 5) // 4, quad_body, 0)
        jb = NCH - 5
        slot(jb + 1, 1, 1, first=False, last=False)
        slot(jb + 2, 0, 2, first=False, last=False)
        slot(jb + 3, 1, 3, first=False, last=False, prefetch=False)
        slot(jb + 4, 0, 0, first=False, last=True, prefetch=False)

        plsc.subcore_barrier()
        pltpu.sync_copy(agg_s.at[pl.ds(tid * RPT, RPT)],
                        aggp_hbm.at[cid, half, pl.ds(tid * RPT, RPT)])
        plsc.subcore_barrier()


def _k3(rec, h0, h1, zrows):
    mesh = plsc.VectorSubcoreMesh(core_axis_name="c", subcore_axis_name="s")
    f = functools.partial(
        pl.kernel,
        mesh=mesh,
        compiler_params=pltpu.CompilerParams(needs_layout_passes=False),
        out_type=jax.ShapeDtypeStruct((2, 2, NP, 128), jnp.float32),
        scratch_types=[
            pltpu.VMEM((3, CH), jnp.int32),
            pltpu.VMEM((3, CH), jnp.int32),
            pltpu.VMEM((3, CH), jnp.int32),
            pltpu.VMEM((3, CH), jnp.int32),
            pltpu.VMEM((CH, 128), jnp.float32),
            pltpu.VMEM((CH, 128), jnp.float32),
            pltpu.SemaphoreType.DMA,
            pltpu.SemaphoreType.DMA,
            pltpu.SemaphoreType.DMA,
            pltpu.SemaphoreType.DMA,
            pltpu.SemaphoreType.DMA,
            pltpu.SemaphoreType.DMA,
            pltpu.VMEM_SHARED((NP, 128), jnp.float32),
        ],
    )(_k3_body)
    return f(rec, h0, h1, zrows)


def _k3c_body(rec_hbm, ea4_hbm, zrows_hbm, wsump_hbm,
              rec0, rec1, rec2, rec3, ea_0, ea_1, eb0, eb1,
              sr0, sr1, se0, se1, ss0, ss1, ws_s):
    cid = lax.axis_index("c")
    sid = lax.axis_index("s")
    wid = sid * 2 + cid
    tid = sid
    recs = (rec0, rec1, rec2, rec3)
    eas = (ea_0, ea_1)
    ebs = (eb0, eb1)
    srs = (sr0, sr1)
    ses = (se0, se1)
    sss = (ss0, ss1)

    def zb(i, c):
        for q in range(8):
            eb0[i, pl.ds(q * 16, 16)] = jnp.zeros((16,), jnp.float32)
            eb1[i, pl.ds(q * 16, 16)] = jnp.zeros((16,), jnp.float32)
        return c

    lax.fori_loop(0, CH, zb, 0)
    pltpu.sync_copy(zrows_hbm, ws_s.at[pl.ds(tid * RPT, RPT)])
    plsc.subcore_barrier()

    def slot(j, b, r, first, second, last):
        if not first:
            pltpu.make_async_copy(rec_hbm.at[wid, j], recs[r], srs[b]).wait()
            pltpu.make_async_copy(ea4_hbm.at[wid, j], eas[b], ses[b]).wait()
        if not last:
            pltpu.async_copy(rec_hbm.at[wid, j + 1], recs[(r + 1) % 4],
                             srs[1 - b])
            pltpu.async_copy(ea4_hbm.at[wid, j + 1], eas[1 - b], ses[1 - b])
        if not (first or second):
            pltpu.make_async_copy(ebs[b], ws_s.at[recs[(r + 2) % 4].at[1]],
                                  sss[b]).wait()

        def grp_body(g, cc):
            av = plsc.bitcast(recs[r][2, pl.ds(g * 16, 16)], jnp.float32)
            for ri in range(16):
                a_s = av[ri]
                rr = g * 16 + ri
                ebs[b][rr, pl.ds(0, 16)] = eas[b][rr, pl.ds(0, 16)] * a_s
            return cc

        lax.fori_loop(0, CH // 16, grp_body, 0)
        if last:
            pltpu.sync_copy(ebs[b], ws_s.at[recs[r].at[1]], add=True)
        else:
            pltpu.async_copy(ebs[b], ws_s.at[recs[r].at[1]], sss[b], add=True)

    pltpu.sync_copy(rec_hbm.at[wid, 0], recs[0])
    pltpu.sync_copy(ea4_hbm.at[wid, 0], eas[0])
    slot(0, 0, 0, first=True, second=False, last=False)
    slot(1, 1, 1, first=False, second=True, last=False)

    def quad_body(t, c):
        jb = 4 * t
        slot(jb + 2, 0, 2, first=False, second=False, last=False)
        slot(jb + 3, 1, 3, first=False, second=False, last=False)
        slot(jb + 4, 0, 0, first=False, second=False, last=False)
        slot(jb + 5, 1, 1, first=False, second=False, last=False)
        return c

    lax.fori_loop(0, (NCH - 5) // 4, quad_body, 0)
    slot(NCH - 3, 0, 2, first=False, second=False, last=False)
    slot(NCH - 2, 1, 3, first=False, second=False, last=False)
    slot(NCH - 1, 0, 0, first=False, second=False, last=True)
    pltpu.make_async_copy(ebs[1], ws_s.at[recs[3].at[1]], sss[1]).wait()

    plsc.subcore_barrier()
    pltpu.sync_copy(ws_s.at[pl.ds(tid * RPT, RPT)],
                    wsump_hbm.at[cid, pl.ds(tid * RPT, RPT)])


def _k3c(rec, ea4, zrows):
    mesh = plsc.VectorSubcoreMesh(core_axis_name="c", subcore_axis_name="s")
    f = functools.partial(
        pl.kernel,
        mesh=mesh,
        compiler_params=pltpu.CompilerParams(needs_layout_passes=False),
        out_type=jax.ShapeDtypeStruct((2, NP, 128), jnp.float32),
        scratch_types=[
            pltpu.VMEM((3, CH), jnp.int32),
            pltpu.VMEM((3, CH), jnp.int32),
            pltpu.VMEM((3, CH), jnp.int32),
            pltpu.VMEM((3, CH), jnp.int32),
            pltpu.VMEM((CH, 16), jnp.float32),
            pltpu.VMEM((CH, 16), jnp.float32),
            pltpu.VMEM((CH, 128), jnp.float32),
            pltpu.VMEM((CH, 128), jnp.float32),
            pltpu.SemaphoreType.DMA,
            pltpu.SemaphoreType.DMA,
            pltpu.SemaphoreType.DMA,
            pltpu.SemaphoreType.DMA,
            pltpu.SemaphoreType.DMA,
            pltpu.SemaphoreType.DMA,
            pltpu.VMEM_SHARED((NP, 128), jnp.float32),
        ],
    )(_k3c_body)
    return f(rec, ea4, zrows)


def _k4a_body(aggp_ref, wsump_ref, h0_ref, h1_ref, den_ref, we_ref, web_ref,
              out_ref):
    aggp = aggp_ref[...]
    agg = jnp.concatenate(
        [aggp[0, 0] + aggp[1, 0], aggp[0, 1] + aggp[1, 1]], axis=-1)
    wsum = (wsump_ref[...][0] + wsump_ref[...][1])[:, :B_IN]
    he = lax.dot_general(wsum, we_ref[...], (((1,), (1,)), ((), ())),
                         preferred_element_type=jnp.float32)
    den = den_ref[...][:, 0]
    asum = den / (den + 1e-16)
    h = jnp.concatenate([h0_ref[...], h1_ref[...]], axis=-1)
    out_ref[...] = jnp.maximum(
        agg + he + asum[:, None] * web_ref[...][None, :] + h, 0.0)


def _k4a(aggp, wsump, h0, h1, denom, conv_We, conv_We_b):
    grid = (N // _RB,)
    return pl.pallas_call(
        _k4a_body,
        grid=grid,
        in_specs=[
            pl.BlockSpec((2, 2, _RB, 128), lambda i: (0, 0, i, 0)),
            pl.BlockSpec((2, _RB, 128), lambda i: (0, i, 0)),
            pl.BlockSpec((_RB, 128), lambda i: (i, 0)),
            pl.BlockSpec((_RB, 128), lambda i: (i, 0)),
            pl.BlockSpec((_RB, 1), lambda i: (i, 0)),
            pl.BlockSpec((DIM, B_IN), lambda i: (0, 0)),
            pl.BlockSpec((DIM,), lambda i: (0,)),
        ],
        out_specs=pl.BlockSpec((_RB, DIM), lambda i: (i, 0)),
        out_shape=jax.ShapeDtypeStruct((N, DIM), jnp.float32),
    )(aggp, wsump, h0, h1, denom.reshape(N, 1), conv_We, conv_We_b)


def _k4b_body(out_ref, batch_ref, fp_ref, fc1W_ref, fc1b_ref, bng_ref,
              bnb_ref, wih_ref, whh_ref, bih_ref, bhh_ref, lin1W_ref,
              lin1b_ref, lin2W_ref, lin2b_ref, res_ref):
    outm = out_ref[...]
    bvec = batch_ref[...]
    gid = lax.broadcasted_iota(jnp.int32, (N, B), 1)
    mask = bvec[:, None] == gid

    q_star = jnp.zeros((B, 2 * DIM), jnp.float32)
    h_l = jnp.zeros((B, DIM), jnp.float32)
    c_l = jnp.zeros((B, DIM), jnp.float32)
    wih = wih_ref[...]
    whh = whh_ref[...]
    bih = bih_ref[...]
    bhh = bhh_ref[...]
    for _ in range(3):
        gates = (lax.dot_general(q_star, wih, (((1,), (1,)), ((), ())),
                                 preferred_element_type=jnp.float32)
                 + bih[None, :]
                 + lax.dot_general(h_l, whh, (((1,), (1,)), ((), ())),
                                   preferred_element_type=jnp.float32)
                 + bhh[None, :])
        i_g = gates[:, :DIM]
        f_g = gates[:, DIM:2 * DIM]
        g_g = gates[:, 2 * DIM:3 * DIM]
        o_g = gates[:, 3 * DIM:]
        c_l = jax.nn.sigmoid(f_g) * c_l + jax.nn.sigmoid(i_g) * jnp.tanh(g_g)
        h_l = jax.nn.sigmoid(o_g) * jnp.tanh(c_l)
        q = h_l
        m = lax.dot_general(outm, q, (((1,), (1,)), ((), ())),
                            preferred_element_type=jnp.float32)
        emax = jnp.max(jnp.where(mask, m, -1e30), axis=0)
        anum = jnp.where(mask, jnp.exp(m - emax[None, :]), 0.0)
        den = jnp.sum(anum, axis=0)
        amat = anum / (den + 1e-16)[None, :]
        r = lax.dot_general(amat, outm, (((0,), (0,)), ((), ())),
                            preferred_element_type=jnp.float32)
        q_star = jnp.concatenate([q, r], axis=-1)

    g_out = jnp.maximum(
        lax.dot_general(q_star, lin1W_ref[...], (((1,), (1,)), ((), ())),
                        preferred_element_type=jnp.float32)
        + lin1b_ref[...][None, :], 0.0)

    h_fp = lax.dot_general(fp_ref[...], fc1W_ref[...], (((1,), (1,)), ((), ())),
                           preferred_element_type=jnp.float32) + fc1b_ref[...][None, :]
    h_fp = h_fp / jnp.sqrt(1.0 + 1e-5) * bng_ref[...][None, :] + bnb_ref[...][None, :]
    out_fp = jnp.where(h_fp > 0, h_fp, jnp.exp(h_fp) - 1.0)

    cat = jnp.concatenate([g_out, out_fp], axis=-1)
    res_ref[...] = (lax.dot_general(cat, lin2W_ref[...], (((1,), (1,)), ((), ())),
                                    preferred_element_type=jnp.float32)
                    + lin2b_ref[...][None, :])


def _k4b(out, batch, fp, fc1_W, fc1_b, bn_g, bn_b, lstm_W_ih, lstm_W_hh,
         lstm_b_ih, lstm_b_hh, lin1_W, lin1_b, lin2_W, lin2_b):
    return pl.pallas_call(
        _k4b_body,
        out_shape=jax.ShapeDtypeStruct((B, OUT), jnp.float32),
    )(out, batch, fp, fc1_W, fc1_b, bn_g, bn_b, lstm_W_ih, lstm_W_hh,
      lstm_b_ih, lstm_b_hh, lin1_W, lin1_b, lin2_W, lin2_b)


def kernel(fp, x, edge_attr, edge_index, batch, fc1_W, fc1_b, bn_g, bn_b,
           lin0_W, lin0_b, conv_W, conv_b, conv_We, conv_We_b,
           a_src, a_dst, a_e, lstm_W_ih, lstm_W_hh, lstm_b_ih, lstm_b_hh,
           lin1_W, lin1_b, lin2_W, lin2_b):
    src = edge_index[0]
    dst = edge_index[1]

    h0, h1, hs2, hd2 = _k0a(x, lin0_W, lin0_b, conv_W, conv_b, a_src, a_dst)
    hs = hs2.reshape(N)
    hd = hd2.reshape(N)
    ea = _k0b(edge_attr, conv_We, a_e, conv_We_b).reshape(E)
    expv, denp = _k1(src, dst, ea, hs, hd)
    denom = _k2(denp.reshape(NW, N))
    alpha = _k2b(dst, expv, denom)

    rec = jnp.stack(
        [src.reshape(NW, NCH, CH),
         dst.reshape(NW, NCH, CH),
         lax.bitcast_convert_type(alpha, jnp.int32).reshape(NW, NCH, CH)],
        axis=2)
    ea4 = jnp.pad(edge_attr, ((0, 0), (0, 16 - B_IN))).reshape(NW, NCH, CH, 16)
    zrows = jnp.zeros((RPT, 128), jnp.float32)

    aggp = _k3(rec, h0, h1, zrows)
    wsump = _k3c(rec, ea4, zrows)
    out = _k4a(aggp, wsump, h0, h1, denom, conv_We, conv_We_b)
    return _k4b(out, batch, fp, fc1_W, fc1_b, bn_g, bn_b, lstm_W_ih,
                lstm_W_hh, lstm_b_ih, lstm_b_hh, lin1_W, lin1_b, lin2_W,
                lin2_b)
